# Initial kernel scaffold; baseline (speedup 1.0000x reference)
import jax
import jax.numpy as jnp
from jax.experimental import pallas as pl


def kernel(x, edge_index, eps, W1, b1, W2, b2, W3, b3):
    raise NotImplementedError("write your pallas kernel here")


# speedup vs baseline: 5.2441x; 5.2441x over previous
import functools

import jax
import jax.numpy as jnp
from jax import lax
from jax.experimental import pallas as pl
from jax.experimental.pallas import tpu as pltpu
from jax.experimental.pallas import tpu_sc as plsc

NC = 2
NS = 16
L = 16

SUB = 128
IDXROWS = 8
PAD_ROWS = 16


@functools.lru_cache(maxsize=None)
def _make_spmm(N, E_pad, HD):
    EPW = E_pad // NS
    NOUTER = EPW // (IDXROWS * SUB)
    ROWS_PAD = N + PAD_ROWS
    RPI = -(-ROWS_PAD // NS // 8) * 8
    RPI_LAST = ROWS_PAD - (NS - 1) * RPI
    RPO = -(-N // NS // 8) * 8
    RPO_LAST = N - (NS - 1) * RPO
    mesh = plsc.VectorSubcoreMesh(core_axis_name="c", subcore_axis_name="s")

    @functools.partial(
        pl.kernel,
        out_type=jax.ShapeDtypeStruct((NC * N, HD), jnp.float32),
        mesh=mesh,
        scratch_types=[
            pltpu.VMEM((IDXROWS, SUB), jnp.int32),
            pltpu.VMEM((IDXROWS, SUB), jnp.int32),
            pltpu.VMEM((SUB, HD), jnp.float32),
            pltpu.VMEM_SHARED((ROWS_PAD, HD), jnp.float32),
        ],
    )
    def spmm(h_hbm, srcm_hbm, dstm_hbm, zeros_hbm, out_hbm,
             src_v, dst_v, rows_v, acc_s):
        cid = lax.axis_index("c")
        sid = lax.axis_index("s")
        cN = cid * N

        @pl.when(sid < NS - 1)
        def _():
            pltpu.sync_copy(zeros_hbm.at[pl.ds(sid * RPI, RPI)],
                            acc_s.at[pl.ds(sid * RPI, RPI)])

        @pl.when(sid == NS - 1)
        def _():
            pltpu.sync_copy(zeros_hbm.at[pl.ds((NS - 1) * RPI, RPI_LAST)],
                            acc_s.at[pl.ds((NS - 1) * RPI, RPI_LAST)])

        plsc.subcore_barrier()

        def chunk_body(ci, carry):
            rbase = sid * (EPW // SUB) + ci * IDXROWS
            pltpu.sync_copy(srcm_hbm.at[pl.ds(rbase, IDXROWS)], src_v)
            pltpu.sync_copy(dstm_hbm.at[pl.ds(rbase, IDXROWS)], dst_v)
            for j in range(IDXROWS):
                for k in range(SUB // L):
                    sl = pl.ds(k * L, L)
                    src_v[j, sl] = src_v[j, sl] + cN
            for j in range(IDXROWS):
                pltpu.sync_copy(h_hbm.at[src_v.at[j]], rows_v)
                pltpu.sync_copy(rows_v, acc_s.at[dst_v.at[j]], add=True)
            return carry

        lax.fori_loop(0, NOUTER, chunk_body, 0)
        plsc.subcore_barrier()

        @pl.when(sid < NS - 1)
        def _():
            pltpu.sync_copy(acc_s.at[pl.ds(sid * RPO, RPO)],
                            out_hbm.at[pl.ds(cN + sid * RPO, RPO)])

        @pl.when(sid == NS - 1)
        def _():
            pltpu.sync_copy(acc_s.at[pl.ds((NS - 1) * RPO, RPO_LAST)],
                            out_hbm.at[pl.ds(cN + (NS - 1) * RPO, RPO_LAST)])

    return spmm


def _make_gemm(N, Hout, relu, BM=1000):
    NB = N // BM
    HB = Hout // 128

    def kern(scale_ref, s0, s1, h0, h1, w_ref, b_ref, o_ref):
        sc = scale_ref[0, 0]
        a0 = h0[...] * sc + s0[...]
        a1 = h1[...] * sc + s1[...]
        z = (jnp.dot(a0, w_ref[:128, :], preferred_element_type=jnp.float32)
             + jnp.dot(a1, w_ref[128:, :], preferred_element_type=jnp.float32)
             + b_ref[...])
        if relu:
            z = jnp.maximum(z, 0.0)
        o_ref[...] = z

    return pl.pallas_call(
        kern,
        grid=(NB, HB),
        in_specs=[
            pl.BlockSpec(memory_space=pltpu.SMEM),
            pl.BlockSpec((BM, 128), lambda i, j: (i, 0)),
            pl.BlockSpec((BM, 128), lambda i, j: (i + NB, 0)),
            pl.BlockSpec((BM, 128), lambda i, j: (i, 0)),
            pl.BlockSpec((BM, 128), lambda i, j: (i + NB, 0)),
            pl.BlockSpec((256, 128), lambda i, j: (0, j)),
            pl.BlockSpec((1, 128), lambda i, j: (0, j)),
        ],
        out_specs=pl.BlockSpec((BM, 128), lambda i, j: (j * NB + i, 0)),
        out_shape=jax.ShapeDtypeStruct((HB * N, 128), jnp.float32),
    )


def kernel(x, edge_index, eps, W1, b1, W2, b2, W3, b3):
    N, D = x.shape
    E = edge_index.shape[1]
    HD = D // 2

    x2 = jnp.concatenate([x[:, :HD], x[:, HD:]], axis=0)

    src = edge_index[0].astype(jnp.int32)
    dst = edge_index[1].astype(jnp.int32)
    EALIGN = NS * IDXROWS * SUB
    E_pad = ((E + EALIGN - 1) // EALIGN) * EALIGN
    pad = E_pad - E
    if pad:
        pi = jnp.arange(pad, dtype=jnp.int32)
        src = jnp.concatenate([src, pi % jnp.int32(N)])
        dst = jnp.concatenate([dst, jnp.int32(N) + (pi % PAD_ROWS)])
    srcm = src.reshape(E_pad // SUB, SUB)
    dstm = dst.reshape(E_pad // SUB, SUB)
    zeros = jnp.zeros((N + PAD_ROWS, HD), jnp.float32)

    spmm = _make_spmm(N, E_pad, HD)
    gemm_h1 = _make_gemm(N, W1.shape[1], relu=True)
    gemm_h2 = _make_gemm(N, W2.shape[1], relu=True)
    gemm_z = _make_gemm(N, W3.shape[1], relu=False)

    scales = (1.0 + eps).reshape(-1, 1, 1)

    s = spmm(x2, srcm, dstm, zeros)
    h = gemm_h1(scales[0], s, s, x2, x2, W1, b1.reshape(1, -1))
    s = spmm(h, srcm, dstm, zeros)
    h = gemm_h2(scales[1], s, s, h, h, W2, b2.reshape(1, -1))
    s = spmm(h, srcm, dstm, zeros)
    z = gemm_z(scales[2], s, s, h, h, W3, b3.reshape(1, -1))
    return z


# speedup vs baseline: 6.9129x; 1.3182x over previous
import functools

import jax
import jax.numpy as jnp
from jax import lax
from jax.experimental import pallas as pl
from jax.experimental.pallas import tpu as pltpu
from jax.experimental.pallas import tpu_sc as plsc

NC = 2
NS = 16
L = 16

SUB = 128
BLKROWS = 40
PAD_ROWS = 16


@functools.lru_cache(maxsize=None)
def _make_spmm(N, E_pad, HD):
    EPW = E_pad // NS
    NSTEP = EPW // SUB
    NBLK = NSTEP // BLKROWS
    EROWS = E_pad // SUB
    ROWS_PAD = N + PAD_ROWS
    RPI = -(-ROWS_PAD // NS // 8) * 8
    RPI_LAST = ROWS_PAD - (NS - 1) * RPI
    RPO = -(-N // NS // 8) * 8
    RPO_LAST = N - (NS - 1) * RPO
    mesh = plsc.VectorSubcoreMesh(core_axis_name="c", subcore_axis_name="s")

    @functools.partial(
        pl.kernel,
        out_type=jax.ShapeDtypeStruct((NC * N, HD), jnp.float32),
        mesh=mesh,
        scratch_types=[
            pltpu.VMEM((BLKROWS, SUB), jnp.int32),
            pltpu.VMEM((BLKROWS, SUB), jnp.int32),
            pltpu.VMEM((SUB, HD), jnp.float32),
            pltpu.VMEM((SUB, HD), jnp.float32),
            pltpu.VMEM_SHARED((ROWS_PAD, HD), jnp.float32),
            pltpu.SemaphoreType.DMA,
            pltpu.SemaphoreType.DMA,
            pltpu.SemaphoreType.DMA,
            pltpu.SemaphoreType.DMA,
        ],
    )
    def spmm(h_hbm, srcm_hbm, dstm_hbm, zeros_hbm, out_hbm,
             src_v, dst_v, rows0, rows1, acc_s,
             gsem0, gsem1, ssem0, ssem1):
        cid = lax.axis_index("c")
        sid = lax.axis_index("s")
        cN = cid * N

        @pl.when(sid < NS - 1)
        def _():
            pltpu.sync_copy(zeros_hbm.at[pl.ds(sid * RPI, RPI)],
                            acc_s.at[pl.ds(sid * RPI, RPI)])

        @pl.when(sid == NS - 1)
        def _():
            pltpu.sync_copy(zeros_hbm.at[pl.ds((NS - 1) * RPI, RPI_LAST)],
                            acc_s.at[pl.ds((NS - 1) * RPI, RPI_LAST)])

        plsc.subcore_barrier()

        def gath(j, rows, sem, issue):
            d = (pltpu.async_copy if issue else pltpu.make_async_copy)(
                h_hbm.at[src_v.at[j]], rows, sem)
            if not issue:
                d.wait()

        def scat(j, rows, sem, issue):
            if issue:
                pltpu.async_copy(rows, acc_s.at[dst_v.at[j]], sem, add=True)
            else:
                pltpu.make_async_copy(rows, acc_s.at[dst_v.at[j]],
                                      sem).wait()

        for b in range(NBLK):
            rb = sid * NSTEP + b * BLKROWS
            pltpu.sync_copy(srcm_hbm.at[pl.ds(cid * EROWS + rb, BLKROWS)],
                            src_v)
            pltpu.sync_copy(dstm_hbm.at[pl.ds(rb, BLKROWS)], dst_v)
            gath(0, rows0, gsem0, True)

            def step_pair(i, carry):
                j0 = 2 * i
                gath(j0, rows0, gsem0, False)
                scat(j0, rows0, ssem0, True)

                @pl.when(i > 0)
                def _():
                    scat(j0 - 1, rows1, ssem1, False)

                gath(j0 + 1, rows1, gsem1, True)
                gath(j0 + 1, rows1, gsem1, False)
                scat(j0 + 1, rows1, ssem1, True)
                scat(j0, rows0, ssem0, False)

                @pl.when(i < BLKROWS // 2 - 1)
                def _():
                    gath(j0 + 2, rows0, gsem0, True)

                return carry

            lax.fori_loop(0, BLKROWS // 2, step_pair, 0)
            scat(BLKROWS - 1, rows1, ssem1, False)

        plsc.subcore_barrier()

        @pl.when(sid < NS - 1)
        def _():
            pltpu.sync_copy(acc_s.at[pl.ds(sid * RPO, RPO)],
                            out_hbm.at[pl.ds(cN + sid * RPO, RPO)])

        @pl.when(sid == NS - 1)
        def _():
            pltpu.sync_copy(acc_s.at[pl.ds((NS - 1) * RPO, RPO_LAST)],
                            out_hbm.at[pl.ds(cN + (NS - 1) * RPO, RPO_LAST)])

    return spmm


def _make_gemm(N, Hout, relu, BM=1000):
    NB = N // BM
    HB = Hout // 128

    def kern(scale_ref, s0, s1, h0, h1, w_ref, b_ref, o_ref):
        sc = scale_ref[0, 0]
        a0 = h0[...] * sc + s0[...]
        a1 = h1[...] * sc + s1[...]
        z = (jnp.dot(a0, w_ref[:128, :], preferred_element_type=jnp.float32)
             + jnp.dot(a1, w_ref[128:, :], preferred_element_type=jnp.float32)
             + b_ref[...])
        if relu:
            z = jnp.maximum(z, 0.0)
        o_ref[...] = z

    return pl.pallas_call(
        kern,
        grid=(NB, HB),
        in_specs=[
            pl.BlockSpec(memory_space=pltpu.SMEM),
            pl.BlockSpec((BM, 128), lambda i, j: (i, 0)),
            pl.BlockSpec((BM, 128), lambda i, j: (i + NB, 0)),
            pl.BlockSpec((BM, 128), lambda i, j: (i, 0)),
            pl.BlockSpec((BM, 128), lambda i, j: (i + NB, 0)),
            pl.BlockSpec((256, 128), lambda i, j: (0, j)),
            pl.BlockSpec((1, 128), lambda i, j: (0, j)),
        ],
        out_specs=pl.BlockSpec((BM, 128), lambda i, j: (j * NB + i, 0)),
        out_shape=jax.ShapeDtypeStruct((HB * N, 128), jnp.float32),
    )


def kernel(x, edge_index, eps, W1, b1, W2, b2, W3, b3):
    N, D = x.shape
    E = edge_index.shape[1]
    HD = D // 2

    x2 = jnp.concatenate([x[:, :HD], x[:, HD:]], axis=0)

    src = edge_index[0].astype(jnp.int32)
    dst = edge_index[1].astype(jnp.int32)
    EALIGN = NS * BLKROWS * SUB
    E_pad = ((E + EALIGN - 1) // EALIGN) * EALIGN
    pad = E_pad - E
    if pad:
        pi = jnp.arange(pad, dtype=jnp.int32)
        src = jnp.concatenate([src, pi % jnp.int32(N)])
        dst = jnp.concatenate([dst, jnp.int32(N) + (pi % PAD_ROWS)])
    srcm = jnp.concatenate([src, src + jnp.int32(N)]).reshape(-1, SUB)
    dstm = dst.reshape(E_pad // SUB, SUB)
    zeros = jnp.zeros((N + PAD_ROWS, HD), jnp.float32)

    spmm = _make_spmm(N, E_pad, HD)
    gemm_h1 = _make_gemm(N, W1.shape[1], relu=True)
    gemm_h2 = _make_gemm(N, W2.shape[1], relu=True)
    gemm_z = _make_gemm(N, W3.shape[1], relu=False)

    scales = (1.0 + eps).reshape(-1, 1, 1)

    s = spmm(x2, srcm, dstm, zeros)
    h = gemm_h1(scales[0], s, s, x2, x2, W1, b1.reshape(1, -1))
    s = spmm(h, srcm, dstm, zeros)
    h = gemm_h2(scales[1], s, s, h, h, W2, b2.reshape(1, -1))
    s = spmm(h, srcm, dstm, zeros)
    z = gemm_z(scales[2], s, s, h, h, W3, b3.reshape(1, -1))
    return z


# speedup vs baseline: 7.0672x; 1.0223x over previous
import functools

import jax
import jax.numpy as jnp
from jax import lax
from jax.experimental import pallas as pl
from jax.experimental.pallas import tpu as pltpu
from jax.experimental.pallas import tpu_sc as plsc

NC = 2
NS = 16
L = 16

SUB = 128
GSUB = 64
BLKROWS = 40
PAD_ROWS = 16


@functools.lru_cache(maxsize=None)
def _make_spmm(N, E_pad, HD):
    EPW = E_pad // NS
    NSTEP = EPW // SUB
    NBLK = NSTEP // BLKROWS
    EROWS = E_pad // SUB
    ROWS_PAD = N + PAD_ROWS
    RPI = -(-ROWS_PAD // NS // 8) * 8
    RPI_LAST = ROWS_PAD - (NS - 1) * RPI
    RPO = -(-N // NS // 8) * 8
    RPO_LAST = N - (NS - 1) * RPO
    mesh = plsc.VectorSubcoreMesh(core_axis_name="c", subcore_axis_name="s")

    @functools.partial(
        pl.kernel,
        out_type=jax.ShapeDtypeStruct((NC * N, HD), jnp.float32),
        mesh=mesh,
        scratch_types=[
            pltpu.VMEM((BLKROWS, SUB), jnp.int32),
            pltpu.VMEM((BLKROWS, SUB), jnp.int32),
            pltpu.VMEM((4 * GSUB, HD), jnp.float32),
            pltpu.VMEM_SHARED((ROWS_PAD, HD), jnp.float32),
            pltpu.SemaphoreType.DMA,
            pltpu.SemaphoreType.DMA,
            pltpu.SemaphoreType.DMA,
            pltpu.SemaphoreType.DMA,
            pltpu.SemaphoreType.DMA,
            pltpu.SemaphoreType.DMA,
        ],
    )
    def spmm(h_hbm, srcm_hbm, dstm_hbm, zeros_hbm, out_hbm,
             src_v, dst_v, arena, acc_s,
             qsem0, qsem1, qsem2, qsem3, ssem0, ssem1):
        cid = lax.axis_index("c")
        sid = lax.axis_index("s")
        cN = cid * N

        @pl.when(sid < NS - 1)
        def _():
            pltpu.sync_copy(zeros_hbm.at[pl.ds(sid * RPI, RPI)],
                            acc_s.at[pl.ds(sid * RPI, RPI)])

        @pl.when(sid == NS - 1)
        def _():
            pltpu.sync_copy(zeros_hbm.at[pl.ds((NS - 1) * RPI, RPI_LAST)],
                            acc_s.at[pl.ds((NS - 1) * RPI, RPI_LAST)])

        plsc.subcore_barrier()

        qsems = (qsem0, qsem1, qsem2, qsem3)

        def gath(r, h, issue, par):
            q = 2 * par + h
            d = (pltpu.async_copy if issue else pltpu.make_async_copy)(
                h_hbm.at[src_v.at[r, pl.ds(h * GSUB, GSUB)]],
                arena.at[pl.ds(q * GSUB, GSUB)], qsems[q])
            if not issue:
                d.wait()

        def scat(k, issue, half):
            sem = ssem0 if half == 0 else ssem1
            if issue:
                pltpu.async_copy(arena.at[pl.ds(half * SUB, SUB)],
                                 acc_s.at[dst_v.at[k]], sem, add=True)
            else:
                pltpu.make_async_copy(arena.at[pl.ds(half * SUB, SUB)],
                                      acc_s.at[dst_v.at[k]], sem).wait()

        for b in range(NBLK):
            rb = sid * NSTEP + b * BLKROWS
            pltpu.sync_copy(srcm_hbm.at[pl.ds(cid * EROWS + rb, BLKROWS)],
                            src_v)
            pltpu.sync_copy(dstm_hbm.at[pl.ds(rb, BLKROWS)], dst_v)
            gath(0, 0, True, 0)
            gath(0, 1, True, 0)
            gath(1, 0, True, 1)
            gath(1, 1, True, 1)

            def step_pair(i, carry):
                kA = 2 * i
                gath(kA, 0, False, 0)
                gath(kA, 1, False, 0)

                @pl.when(i > 0)
                def _():
                    scat(kA - 1, False, 1)
                    gath(kA + 1, 0, True, 1)
                    gath(kA + 1, 1, True, 1)

                scat(kA, True, 0)
                gath(kA + 1, 0, False, 1)
                gath(kA + 1, 1, False, 1)
                scat(kA, False, 0)

                @pl.when(i < BLKROWS // 2 - 1)
                def _():
                    gath(kA + 2, 0, True, 0)
                    gath(kA + 2, 1, True, 0)

                scat(kA + 1, True, 1)
                return carry

            lax.fori_loop(0, BLKROWS // 2, step_pair, 0)
            scat(BLKROWS - 1, False, 1)

        plsc.subcore_barrier()

        @pl.when(sid < NS - 1)
        def _():
            pltpu.sync_copy(acc_s.at[pl.ds(sid * RPO, RPO)],
                            out_hbm.at[pl.ds(cN + sid * RPO, RPO)])

        @pl.when(sid == NS - 1)
        def _():
            pltpu.sync_copy(acc_s.at[pl.ds((NS - 1) * RPO, RPO_LAST)],
                            out_hbm.at[pl.ds(cN + (NS - 1) * RPO, RPO_LAST)])

    return spmm


def _make_gemm(N, Hout, relu, BM=1000):
    NB = N // BM
    HB = Hout // 128

    def kern(scale_ref, s0, s1, h0, h1, w_ref, b_ref, o_ref):
        sc = scale_ref[0, 0]
        a0 = h0[...] * sc + s0[...]
        a1 = h1[...] * sc + s1[...]
        z = (jnp.dot(a0, w_ref[:128, :], preferred_element_type=jnp.float32)
             + jnp.dot(a1, w_ref[128:, :], preferred_element_type=jnp.float32)
             + b_ref[...])
        if relu:
            z = jnp.maximum(z, 0.0)
        o_ref[...] = z

    return pl.pallas_call(
        kern,
        grid=(NB, HB),
        in_specs=[
            pl.BlockSpec(memory_space=pltpu.SMEM),
            pl.BlockSpec((BM, 128), lambda i, j: (i, 0)),
            pl.BlockSpec((BM, 128), lambda i, j: (i + NB, 0)),
            pl.BlockSpec((BM, 128), lambda i, j: (i, 0)),
            pl.BlockSpec((BM, 128), lambda i, j: (i + NB, 0)),
            pl.BlockSpec((256, 128), lambda i, j: (0, j)),
            pl.BlockSpec((1, 128), lambda i, j: (0, j)),
        ],
        out_specs=pl.BlockSpec((BM, 128), lambda i, j: (j * NB + i, 0)),
        out_shape=jax.ShapeDtypeStruct((HB * N, 128), jnp.float32),
    )


def kernel(x, edge_index, eps, W1, b1, W2, b2, W3, b3):
    N, D = x.shape
    E = edge_index.shape[1]
    HD = D // 2

    x2 = jnp.concatenate([x[:, :HD], x[:, HD:]], axis=0)

    src = edge_index[0].astype(jnp.int32)
    dst = edge_index[1].astype(jnp.int32)
    EALIGN = NS * BLKROWS * SUB
    E_pad = ((E + EALIGN - 1) // EALIGN) * EALIGN
    pad = E_pad - E
    if pad:
        pi = jnp.arange(pad, dtype=jnp.int32)
        src = jnp.concatenate([src, pi % jnp.int32(N)])
        dst = jnp.concatenate([dst, jnp.int32(N) + (pi % PAD_ROWS)])
    srcm = jnp.concatenate([src, src + jnp.int32(N)]).reshape(-1, SUB)
    dstm = dst.reshape(E_pad // SUB, SUB)
    zeros = jnp.zeros((N + PAD_ROWS, HD), jnp.float32)

    spmm = _make_spmm(N, E_pad, HD)
    gemm_h1 = _make_gemm(N, W1.shape[1], relu=True)
    gemm_h2 = _make_gemm(N, W2.shape[1], relu=True)
    gemm_z = _make_gemm(N, W3.shape[1], relu=False)

    scales = (1.0 + eps).reshape(-1, 1, 1)

    s = spmm(x2, srcm, dstm, zeros)
    h = gemm_h1(scales[0], s, s, x2, x2, W1, b1.reshape(1, -1))
    s = spmm(h, srcm, dstm, zeros)
    h = gemm_h2(scales[1], s, s, h, h, W2, b2.reshape(1, -1))
    s = spmm(h, srcm, dstm, zeros)
    z = gemm_z(scales[2], s, s, h, h, W3, b3.reshape(1, -1))
    return z


# speedup vs baseline: 7.3220x; 1.0361x over previous
import functools

import jax
import jax.numpy as jnp
from jax import lax
from jax.experimental import pallas as pl
from jax.experimental.pallas import tpu as pltpu
from jax.experimental.pallas import tpu_sc as plsc

NC = 2
NS = 16
L = 16

SUB = 128
GSUB = 64
BLKROWS = 40
PAD_ROWS = 16


@functools.lru_cache(maxsize=None)
def _make_spmm(N, E_pad, HD):
    EPW = E_pad // NS
    NSTEP = EPW // SUB
    NBLK = NSTEP // BLKROWS
    EROWS = E_pad // SUB
    ROWS_PAD = N + PAD_ROWS
    RPI = -(-ROWS_PAD // NS // 8) * 8
    RPI_LAST = ROWS_PAD - (NS - 1) * RPI
    RPO = -(-N // NS // 8) * 8
    RPO_LAST = N - (NS - 1) * RPO
    mesh = plsc.VectorSubcoreMesh(core_axis_name="c", subcore_axis_name="s")

    @functools.partial(
        pl.kernel,
        out_type=jax.ShapeDtypeStruct((NC * N, HD), jnp.float32),
        mesh=mesh,
        scratch_types=[
            pltpu.VMEM((BLKROWS, SUB), jnp.int32),
            pltpu.VMEM((BLKROWS, SUB), jnp.int32),
            pltpu.VMEM((4 * GSUB, HD), jnp.float32),
            pltpu.VMEM_SHARED((ROWS_PAD, HD), jnp.float32),
            pltpu.SemaphoreType.DMA,
            pltpu.SemaphoreType.DMA,
            pltpu.SemaphoreType.DMA,
            pltpu.SemaphoreType.DMA,
            pltpu.SemaphoreType.DMA,
            pltpu.SemaphoreType.DMA,
        ],
    )
    def spmm(h_hbm, srcm_hbm, dstm_hbm, zeros_hbm, out_hbm,
             src_v, dst_v, arena, acc_s,
             qsem0, qsem1, qsem2, qsem3, ssem0, ssem1):
        cid = lax.axis_index("c")
        sid = lax.axis_index("s")
        cN = cid * N

        @pl.when(sid < NS - 1)
        def _():
            pltpu.sync_copy(zeros_hbm.at[pl.ds(sid * RPI, RPI)],
                            acc_s.at[pl.ds(sid * RPI, RPI)])

        @pl.when(sid == NS - 1)
        def _():
            pltpu.sync_copy(zeros_hbm.at[pl.ds((NS - 1) * RPI, RPI_LAST)],
                            acc_s.at[pl.ds((NS - 1) * RPI, RPI_LAST)])

        plsc.subcore_barrier()

        qsems = (qsem0, qsem1, qsem2, qsem3)

        def gath(r, h, issue, par):
            q = 2 * par + h
            d = (pltpu.async_copy if issue else pltpu.make_async_copy)(
                h_hbm.at[src_v.at[r, pl.ds(h * GSUB, GSUB)]],
                arena.at[pl.ds(q * GSUB, GSUB)], qsems[q])
            if not issue:
                d.wait()

        def scat(k, issue, half):
            sem = ssem0 if half == 0 else ssem1
            if issue:
                pltpu.async_copy(arena.at[pl.ds(half * SUB, SUB)],
                                 acc_s.at[dst_v.at[k]], sem, add=True)
            else:
                pltpu.make_async_copy(arena.at[pl.ds(half * SUB, SUB)],
                                      acc_s.at[dst_v.at[k]], sem).wait()

        for b in range(NBLK):
            rb = sid * NSTEP + b * BLKROWS
            pltpu.sync_copy(srcm_hbm.at[pl.ds(cid * EROWS + rb, BLKROWS)],
                            src_v)
            pltpu.sync_copy(dstm_hbm.at[pl.ds(rb, BLKROWS)], dst_v)
            gath(0, 0, True, 0)
            gath(0, 1, True, 0)
            gath(1, 0, True, 1)
            gath(1, 1, True, 1)

            def step_pair(i, carry):
                kA = 2 * i
                gath(kA, 0, False, 0)
                gath(kA, 1, False, 0)

                @pl.when(i > 0)
                def _():
                    scat(kA - 1, False, 1)
                    gath(kA + 1, 0, True, 1)
                    gath(kA + 1, 1, True, 1)

                scat(kA, True, 0)
                gath(kA + 1, 0, False, 1)
                gath(kA + 1, 1, False, 1)
                scat(kA, False, 0)

                @pl.when(i < BLKROWS // 2 - 1)
                def _():
                    gath(kA + 2, 0, True, 0)
                    gath(kA + 2, 1, True, 0)

                scat(kA + 1, True, 1)
                return carry

            lax.fori_loop(0, BLKROWS // 2, step_pair, 0)
            scat(BLKROWS - 1, False, 1)

        plsc.subcore_barrier()

        @pl.when(sid < NS - 1)
        def _():
            pltpu.sync_copy(acc_s.at[pl.ds(sid * RPO, RPO)],
                            out_hbm.at[pl.ds(cN + sid * RPO, RPO)])

        @pl.when(sid == NS - 1)
        def _():
            pltpu.sync_copy(acc_s.at[pl.ds((NS - 1) * RPO, RPO_LAST)],
                            out_hbm.at[pl.ds(cN + (NS - 1) * RPO, RPO_LAST)])

    return spmm


def _make_gemm(N, Hout, relu, BM=2000):
    NB = N // BM
    HB = Hout // 128

    def kern(scale_ref, s0, s1, h0, h1, w_ref, b_ref, o_ref):
        sc = scale_ref[0, 0]
        a0 = h0[...] * sc + s0[...]
        a1 = h1[...] * sc + s1[...]
        z = (jnp.dot(a0, w_ref[:128, :], preferred_element_type=jnp.float32)
             + jnp.dot(a1, w_ref[128:, :], preferred_element_type=jnp.float32)
             + b_ref[...])
        if relu:
            z = jnp.maximum(z, 0.0)
        o_ref[...] = z

    return pl.pallas_call(
        kern,
        grid=(NB, HB),
        in_specs=[
            pl.BlockSpec(memory_space=pltpu.SMEM),
            pl.BlockSpec((BM, 128), lambda i, j: (i, 0)),
            pl.BlockSpec((BM, 128), lambda i, j: (i + NB, 0)),
            pl.BlockSpec((BM, 128), lambda i, j: (i, 0)),
            pl.BlockSpec((BM, 128), lambda i, j: (i + NB, 0)),
            pl.BlockSpec((256, 128), lambda i, j: (0, j)),
            pl.BlockSpec((1, 128), lambda i, j: (0, j)),
        ],
        out_specs=pl.BlockSpec((BM, 128), lambda i, j: (j * NB + i, 0)),
        out_shape=jax.ShapeDtypeStruct((HB * N, 128), jnp.float32),
        compiler_params=pltpu.CompilerParams(
            dimension_semantics=("arbitrary", "arbitrary")),
    )


def kernel(x, edge_index, eps, W1, b1, W2, b2, W3, b3):
    N, D = x.shape
    E = edge_index.shape[1]
    HD = D // 2

    x2 = jnp.concatenate([x[:, :HD], x[:, HD:]], axis=0)

    src = edge_index[0].astype(jnp.int32)
    dst = edge_index[1].astype(jnp.int32)
    EALIGN = NS * BLKROWS * SUB
    E_pad = ((E + EALIGN - 1) // EALIGN) * EALIGN
    pad = E_pad - E
    if pad:
        pi = jnp.arange(pad, dtype=jnp.int32)
        src = jnp.concatenate([src, pi % jnp.int32(N)])
        dst = jnp.concatenate([dst, jnp.int32(N) + (pi % PAD_ROWS)])
    srcm = jnp.concatenate([src, src + jnp.int32(N)]).reshape(-1, SUB)
    dstm = dst.reshape(E_pad // SUB, SUB)
    zeros = jnp.zeros((N + PAD_ROWS, HD), jnp.float32)

    spmm = _make_spmm(N, E_pad, HD)
    gemm_h1 = _make_gemm(N, W1.shape[1], relu=True)
    gemm_h2 = _make_gemm(N, W2.shape[1], relu=True)
    gemm_z = _make_gemm(N, W3.shape[1], relu=False)

    scales = (1.0 + eps).reshape(-1, 1, 1)

    s = spmm(x2, srcm, dstm, zeros)
    h = gemm_h1(scales[0], s, s, x2, x2, W1, b1.reshape(1, -1))
    s = spmm(h, srcm, dstm, zeros)
    h = gemm_h2(scales[1], s, s, h, h, W2, b2.reshape(1, -1))
    s = spmm(h, srcm, dstm, zeros)
    z = gemm_z(scales[2], s, s, h, h, W3, b3.reshape(1, -1))
    return z


# speedup vs baseline: 7.5281x; 1.0281x over previous
import functools

import jax
import jax.numpy as jnp
from jax import lax
from jax.experimental import pallas as pl
from jax.experimental.pallas import tpu as pltpu
from jax.experimental.pallas import tpu_sc as plsc

NC = 2
NS = 16
L = 16

SUB = 128
GSUB = 64
BLKROWS = 40


@functools.lru_cache(maxsize=None)
def _make_spmm(N, E, HD):
    EROWS = E // SUB
    RPT = -(-EROWS // NS // 8) * 8
    RPT_LAST = EROWS - (NS - 1) * RPT
    RPO = -(-N // NS // 8) * 8
    RPO_LAST = N - (NS - 1) * RPO

    def blocks_of(nrows):
        out, off = [], 0
        while off < nrows:
            nb = min(BLKROWS, nrows - off)
            out.append((off, nb))
            off += nb
        return tuple(out)

    BLOCKS_MAIN = blocks_of(RPT)
    BLOCKS_LAST = blocks_of(RPT_LAST)
    ARENA = 4 * GSUB
    mesh = plsc.VectorSubcoreMesh(core_axis_name="c", subcore_axis_name="s")

    @functools.partial(
        pl.kernel,
        out_type=jax.ShapeDtypeStruct((NC * N, HD), jnp.float32),
        mesh=mesh,
        scratch_types=[
            pltpu.VMEM((BLKROWS, SUB), jnp.int32),
            pltpu.VMEM((BLKROWS, SUB), jnp.int32),
            pltpu.VMEM((ARENA, HD), jnp.float32),
            pltpu.VMEM_SHARED((N, HD), jnp.float32),
            pltpu.SemaphoreType.DMA,
            pltpu.SemaphoreType.DMA,
            pltpu.SemaphoreType.DMA,
            pltpu.SemaphoreType.DMA,
            pltpu.SemaphoreType.DMA,
            pltpu.SemaphoreType.DMA,
        ],
    )
    def spmm(h_hbm, srcm_hbm, dstm_hbm, out_hbm,
             src_v, dst_v, arena, acc_s,
             qsem0, qsem1, qsem2, qsem3, ssem0, ssem1):
        cid = lax.axis_index("c")
        sid = lax.axis_index("s")
        cN = cid * N
        zeros16 = jnp.zeros((L,), jnp.float32)

        def zrow(r, carry):
            for k in range(HD // L):
                arena[r, pl.ds(k * L, L)] = zeros16
            return carry

        lax.fori_loop(0, ARENA, zrow, 0)

        def zinit(base, nrows):
            off = 0
            while off < nrows:
                nn = min(ARENA, nrows - off)
                pltpu.sync_copy(arena.at[pl.ds(0, nn)],
                                acc_s.at[pl.ds(base + off, nn)])
                off += nn

        @pl.when(sid < NS - 1)
        def _():
            zinit(sid * RPO, RPO)

        @pl.when(sid == NS - 1)
        def _():
            zinit((NS - 1) * RPO, RPO_LAST)

        plsc.subcore_barrier()

        qsems = (qsem0, qsem1, qsem2, qsem3)

        def gath(r, h, issue, par):
            q = 2 * par + h
            d = (pltpu.async_copy if issue else pltpu.make_async_copy)(
                h_hbm.at[src_v.at[r, pl.ds(h * GSUB, GSUB)]],
                arena.at[pl.ds(q * GSUB, GSUB)], qsems[q])
            if not issue:
                d.wait()

        def scat(k, issue, half):
            sem = ssem0 if half == 0 else ssem1
            if issue:
                pltpu.async_copy(arena.at[pl.ds(half * SUB, SUB)],
                                 acc_s.at[dst_v.at[k]], sem, add=True)
            else:
                pltpu.make_async_copy(arena.at[pl.ds(half * SUB, SUB)],
                                      acc_s.at[dst_v.at[k]], sem).wait()

        def run_block(rb, nrows):
            pltpu.sync_copy(srcm_hbm.at[pl.ds(rb, nrows)],
                            src_v.at[pl.ds(0, nrows)])
            pltpu.sync_copy(dstm_hbm.at[pl.ds(rb, nrows)],
                            dst_v.at[pl.ds(0, nrows)])

            def addrow(r, carry):
                for k in range(SUB // L):
                    sl = pl.ds(k * L, L)
                    src_v[r, sl] = src_v[r, sl] + cN
                return carry

            lax.fori_loop(0, nrows, addrow, 0)

            gath(0, 0, True, 0)
            gath(0, 1, True, 0)
            gath(1, 0, True, 1)
            gath(1, 1, True, 1)

            def step_pair(i, carry):
                kA = 2 * i
                gath(kA, 0, False, 0)
                gath(kA, 1, False, 0)

                @pl.when(i > 0)
                def _():
                    scat(kA - 1, False, 1)
                    gath(kA + 1, 0, True, 1)
                    gath(kA + 1, 1, True, 1)

                scat(kA, True, 0)
                gath(kA + 1, 0, False, 1)
                gath(kA + 1, 1, False, 1)
                scat(kA, False, 0)

                @pl.when(i < nrows // 2 - 1)
                def _():
                    gath(kA + 2, 0, True, 0)
                    gath(kA + 2, 1, True, 0)

                scat(kA + 1, True, 1)
                return carry

            lax.fori_loop(0, nrows // 2, step_pair, 0)
            scat(nrows - 1, False, 1)

        @pl.when(sid < NS - 1)
        def _():
            for off, nb in BLOCKS_MAIN:
                run_block(sid * RPT + off, nb)

        @pl.when(sid == NS - 1)
        def _():
            for off, nb in BLOCKS_LAST:
                run_block((NS - 1) * RPT + off, nb)

        plsc.subcore_barrier()

        @pl.when(sid < NS - 1)
        def _():
            pltpu.sync_copy(acc_s.at[pl.ds(sid * RPO, RPO)],
                            out_hbm.at[pl.ds(cN + sid * RPO, RPO)])

        @pl.when(sid == NS - 1)
        def _():
            pltpu.sync_copy(acc_s.at[pl.ds((NS - 1) * RPO, RPO_LAST)],
                            out_hbm.at[pl.ds(cN + (NS - 1) * RPO, RPO_LAST)])

    return spmm


def _make_gemm(N, Hout, relu, BM=2000):
    NB = N // BM
    HB = Hout // 128

    def kern(scale_ref, s0, s1, h0, h1, w_ref, b_ref, o_ref):
        sc = scale_ref[0, 0]
        a0 = h0[...] * sc + s0[...]
        a1 = h1[...] * sc + s1[...]
        z = (jnp.dot(a0, w_ref[:128, :], preferred_element_type=jnp.float32)
             + jnp.dot(a1, w_ref[128:, :], preferred_element_type=jnp.float32)
             + b_ref[...])
        if relu:
            z = jnp.maximum(z, 0.0)
        o_ref[...] = z

    return pl.pallas_call(
        kern,
        grid=(NB, HB),
        in_specs=[
            pl.BlockSpec(memory_space=pltpu.SMEM),
            pl.BlockSpec((BM, 128), lambda i, j: (i, 0)),
            pl.BlockSpec((BM, 128), lambda i, j: (i + NB, 0)),
            pl.BlockSpec((BM, 128), lambda i, j: (i, 0)),
            pl.BlockSpec((BM, 128), lambda i, j: (i + NB, 0)),
            pl.BlockSpec((256, 128), lambda i, j: (0, j)),
            pl.BlockSpec((1, 128), lambda i, j: (0, j)),
        ],
        out_specs=pl.BlockSpec((BM, 128), lambda i, j: (j * NB + i, 0)),
        out_shape=jax.ShapeDtypeStruct((HB * N, 128), jnp.float32),
        compiler_params=pltpu.CompilerParams(
            dimension_semantics=("arbitrary", "arbitrary")),
    )


def kernel(x, edge_index, eps, W1, b1, W2, b2, W3, b3):
    N, D = x.shape
    E = edge_index.shape[1]
    HD = D // 2

    x2 = jnp.concatenate([x[:, :HD], x[:, HD:]], axis=0)

    srcm = edge_index[0].astype(jnp.int32).reshape(E // SUB, SUB)
    dstm = edge_index[1].astype(jnp.int32).reshape(E // SUB, SUB)

    spmm = _make_spmm(N, E, HD)
    gemm_h1 = _make_gemm(N, W1.shape[1], relu=True)
    gemm_h2 = _make_gemm(N, W2.shape[1], relu=True)
    gemm_z = _make_gemm(N, W3.shape[1], relu=False)

    scales = (1.0 + eps).reshape(-1, 1, 1)

    s = spmm(x2, srcm, dstm)
    h = gemm_h1(scales[0], s, s, x2, x2, W1, b1.reshape(1, -1))
    s = spmm(h, srcm, dstm)
    h = gemm_h2(scales[1], s, s, h, h, W2, b2.reshape(1, -1))
    s = spmm(h, srcm, dstm)
    z = gemm_z(scales[2], s, s, h, h, W3, b3.reshape(1, -1))
    return z
